# baseline (speedup 1.0000x reference)
import functools

import jax
import jax.numpy as jnp
from jax.experimental import pallas as pl
from jax.experimental.pallas import tpu as pltpu

G = 128
NPG = 78
N = G * NPG
EMB = 128
NUM_LAYERS = 4
K = 16
CUTOFF = 5.0
EPS = 1e-5
NP = 128
NR = 80
GB = 8
GSTEPS = G // GB
BN = GB * NP


def _build_body(posl_ref, poss_ref, sc_ref):
    i_row = jax.lax.broadcasted_iota(jnp.int32, (NP, 1), 0)
    j_col = jax.lax.broadcasted_iota(jnp.int32, (1, NP), 1)
    d2s = []
    for c in range(GB):
        px_r = posl_ref[0, 8 * c + 0, :].reshape(1, NP)
        py_r = posl_ref[0, 8 * c + 1, :].reshape(1, NP)
        pz_r = posl_ref[0, 8 * c + 2, :].reshape(1, NP)
        px_c = poss_ref[0, c * NP:(c + 1) * NP, 0:1]
        py_c = poss_ref[0, c * NP:(c + 1) * NP, 1:2]
        pz_c = poss_ref[0, c * NP:(c + 1) * NP, 2:3]
        dx = px_c - px_r
        dy = py_c - py_r
        dz = pz_c - pz_r
        d2c = (dx * dx + dy * dy) + dz * dz
        d2c = jnp.where(i_row == j_col, d2c + 1e9, d2c)
        d2s.append(d2c)
    d2 = jnp.concatenate(d2s, axis=0)
    jmat = jax.lax.broadcasted_iota(jnp.int32, (BN, NP), 1)
    jrow = jax.lax.broadcasted_iota(jnp.int32, (1, NP), 1)
    d2 = jnp.where(jrow >= NPG, 2e9, d2)
    ib = jax.lax.broadcasted_iota(jnp.int32, (BN, 1), 0)
    row_ok = (jnp.bitwise_and(ib, NP - 1) < NPG).astype(jnp.float32)
    cols, dists, invs = [], [], []
    for _ in range(K):
        mval = jnp.min(d2, axis=1, keepdims=True)
        idx = jnp.min(jnp.where(d2 == mval, jmat, jnp.int32(10**9)),
                      axis=1, keepdims=True)
        cols.append(idx.astype(jnp.float32))
        dists.append(jnp.sqrt(mval))
        invs.append(1.0 - jnp.where(mval <= CUTOFF * CUTOFF, row_ok, 0.0))
        d2 = jnp.where(jmat == idx, 2e9, d2)
    chunks = []
    for c in range(GB):
        r0 = c * NP
        for k in range(K):
            chunks.append(jnp.concatenate(
                [dists[k][r0:r0 + NR], invs[k][r0:r0 + NR],
                 cols[k][r0:r0 + NR], jnp.zeros((NR, 5), jnp.float32)],
                axis=1))
    sc_ref[0] = jnp.concatenate(chunks, axis=0)


def _bn_in(stats_ref, gb_ref):
    s = stats_ref[...]
    mu = s[0:1, :] / N
    var = s[1:2, :] / N - mu * mu
    rstd = jax.lax.rsqrt(var + EPS)
    return mu, rstd, gb_ref[0:1, :], gb_ref[1:2, :]


def _msg_core(xall, sc_ref, wp_ref, wq_ref, wrow_ref, agg_s, st1_s, i):
    pqp = jnp.dot(xall, wp_ref[...],
                  preferred_element_type=jnp.float32)
    pqq = jnp.dot(xall, wq_ref[...],
                  preferred_element_type=jnp.float32)
    jv = jax.lax.broadcasted_iota(jnp.int32, (1, NP), 1)
    wrow = wrow_ref[0:1, :]
    brow = wrow_ref[1:2, :]
    invrow = jnp.full((1, 256), -1e9, jnp.float32)
    s1 = jnp.zeros((1, EMB), jnp.float32)
    s2 = jnp.zeros((1, EMB), jnp.float32)
    EPG = K * NR
    for c in range(GB):
        r0 = c * NP
        pfs = pqp[r0:r0 + NR, :] + brow
        qfs = pqq[r0:r0 + NP, :]
        sc = sc_ref[0, c * EPG:(c + 1) * EPG, :]
        ohs = (sc[:, 2:3].astype(jnp.int32) == jv).astype(jnp.float32)
        comb = jnp.concatenate([ohs, sc[:, 0:2]], axis=1)
        table = jnp.concatenate([qfs, wrow, invrow], axis=0)
        zfs = jnp.dot(comb, table, preferred_element_type=jnp.float32)
        agg = jnp.zeros((NR, EMB), jnp.float32)
        for k in range(K):
            zk = zfs[k * NR:(k + 1) * NR, :] + pfs
            agg = agg + jax.nn.sigmoid(zk[:, 0:128]) \
                * jax.nn.softplus(zk[:, 128:256])
        agg_s[pl.ds(i * BN + r0, NR), :] = agg
        agg_s[pl.ds(i * BN + r0 + NR, NP - NR), :] = \
            jnp.zeros((NP - NR, EMB), jnp.float32)
        s1 = s1 + jnp.sum(agg, axis=0, keepdims=True)
        s2 = s2 + jnp.sum(agg * agg, axis=0, keepdims=True)
    st1_s[...] += jnp.concatenate(
        [s1, s2, jnp.zeros((6, EMB), jnp.float32)], axis=0)


def _layer_impl(first, xz_ref, emb_ref, stats2_ref, g2b2_ref, sc_ref,
                wp_ref, wq_ref, wrow_ref, g1b1_ref, xn_ref, st2_ref,
                agg_s, st1_s):
    i = pl.program_id(0)

    def compute_x():
        if first:
            zc = xz_ref[0, :, 0:1]
            cls = jax.lax.broadcasted_iota(jnp.int32, (1, 128), 1)
            onehot = (zc == cls).astype(jnp.float32)
            return jnp.dot(onehot, emb_ref[...],
                           preferred_element_type=jnp.float32)
        mu, rstd, g2, b2 = _bn_in(stats2_ref, g2b2_ref)
        ib = jax.lax.broadcasted_iota(jnp.int32, (BN, 1), 0)
        mask = (jnp.bitwise_and(ib, NP - 1) < NPG).astype(jnp.float32)
        return jax.nn.relu((xz_ref[0] - mu) * rstd * g2 + b2) * mask

    @pl.when(i == 0)
    def _init():
        st1_s[...] = jnp.zeros((8, EMB), jnp.float32)

    @pl.when(i < GSTEPS)
    def _phase_msg():
        _msg_core(compute_x(), sc_ref, wp_ref, wq_ref, wrow_ref,
                  agg_s, st1_s, i)

    @pl.when(i >= GSTEPS)
    def _phase_bn():
        s = st1_s[...]
        mu1 = s[0:1, :] / N
        var = s[1:2, :] / N - mu1 * mu1
        rstd1 = jax.lax.rsqrt(var + EPS)
        g1 = g1b1_ref[0:1, :]
        b1 = g1b1_ref[1:2, :]
        ib = jax.lax.broadcasted_iota(jnp.int32, (BN, 1), 0)
        mask = (jnp.bitwise_and(ib, NP - 1) < NPG).astype(jnp.float32)
        agg = agg_s[pl.ds((i - GSTEPS) * BN, BN), :]
        xo = (compute_x() + ((agg - mu1) * rstd1 * g1 + b1)) * mask
        xn_ref[0] = xo
        contrib = jnp.concatenate(
            [jnp.sum(xo, axis=0, keepdims=True),
             jnp.sum(xo * xo, axis=0, keepdims=True),
             jnp.zeros((6, EMB), jnp.float32)], axis=0)

        @pl.when(i == GSTEPS)
        def _():
            st2_ref[...] = contrib

        @pl.when(i > GSTEPS)
        def _():
            st2_ref[...] += contrib


def _layer_first(z_ref, emb_ref, sc_ref, wp_ref, wq_ref, wrow_ref,
                 g1b1_ref, xn_ref, st2_ref, agg_s, st1_s):
    _layer_impl(True, z_ref, emb_ref, None, None, sc_ref, wp_ref, wq_ref,
                wrow_ref, g1b1_ref, xn_ref, st2_ref, agg_s, st1_s)


def _layer_rest(xn_in_ref, stats2_ref, g2b2_ref, sc_ref, wp_ref, wq_ref,
                wrow_ref, g1b1_ref, xn_ref, st2_ref, agg_s, st1_s):
    _layer_impl(False, xn_in_ref, None, stats2_ref, g2b2_ref, sc_ref,
                wp_ref, wq_ref, wrow_ref, g1b1_ref, xn_ref, st2_ref,
                agg_s, st1_s)


def _head_body(xn_ref, stats2_ref, g2b2_ref, w1_ref, hw_ref, out_ref):
    mu, rstd, g2, b2 = _bn_in(stats2_ref, g2b2_ref)
    xn = xn_ref[...]
    ir = jax.lax.broadcasted_iota(jnp.int32, (1, NP, 1), 1)
    mask = (ir < NPG).astype(jnp.float32)
    x = jax.nn.relu((xn - mu[None]) * rstd[None] * g2[None] + b2[None]) * mask
    gmean = jnp.sum(x, axis=1) / NPG
    h = jax.nn.relu(
        jnp.dot(gmean, w1_ref[...], preferred_element_type=jnp.float32)
        + hw_ref[1:2, :])
    o = jnp.sum(h * hw_ref[0:1, :], axis=1, keepdims=True) + hw_ref[2:3, 0:1]
    out_ref[...] = jnp.broadcast_to(o, (G, 128))


def _pad_rows(a, rows):
    return jnp.pad(a, ((0, rows - a.shape[0]), (0, 0)))


@functools.partial(jax.jit, static_argnames=())
def kernel(z, pos, batch, params):
    f32 = jnp.float32
    posg = pos.reshape(G, NPG, 3)
    poss = jnp.pad(posg, ((0, 0), (0, NP - NPG), (0, 5))
                   ).reshape(GSTEPS, BN, 8)
    posl = jnp.pad(posg.transpose(0, 2, 1), ((0, 0), (0, 5), (0, NP - NPG))
                   ).reshape(GSTEPS, GB * 8, NP)
    z3 = jnp.pad(z.reshape(G, NPG, 1), ((0, 0), (0, NP - NPG), (0, 7))
                 ).reshape(GSTEPS, BN, 8)
    emb_pad = jnp.pad(params["emb"], ((0, 28), (0, 0)))

    gspec = lambda shape: pl.BlockSpec((1,) + shape, lambda g: (g, 0, 0))
    full2 = lambda a: pl.BlockSpec(a.shape, lambda g: (0,) * a.ndim)
    scspec = gspec((GB * K * NR, 8))

    sc = pl.pallas_call(
        _build_body,
        grid=(GSTEPS,),
        in_specs=[gspec((GB * 8, NP)), gspec((BN, 8))],
        out_specs=scspec,
        out_shape=jax.ShapeDtypeStruct((GSTEPS, GB * K * NR, 8), f32),
    )(posl, poss)

    folded = lambda shape: pl.BlockSpec(
        (1,) + shape,
        lambda i: (jnp.where(i < GSTEPS, i, i - GSTEPS), 0, 0))
    scratch = [pltpu.VMEM((GSTEPS * BN, EMB), f32),
               pltpu.VMEM((8, EMB), f32)]
    layer_out_specs = [folded((BN, EMB)),
                       pl.BlockSpec((8, EMB), lambda i: (0, 0))]
    layer_out_shape = [jax.ShapeDtypeStruct((GSTEPS, BN, EMB), f32),
                       jax.ShapeDtypeStruct((8, EMB), f32)]

    xn, stats2, g2b2 = None, None, None
    for li, layer in enumerate(params["layers"]):
        wf, ws = layer["Wf"], layer["Ws"]
        wcat_p = jnp.concatenate([wf[0:128], ws[0:128]], axis=1)
        wcat_q = jnp.concatenate([wf[128:256], ws[128:256]], axis=1)
        wrow = _pad_rows(jnp.stack([
            jnp.concatenate([wf[256], ws[256]]),
            jnp.concatenate([layer["bf"], layer["bs"]]),
        ]), 8)
        g1b1 = _pad_rows(jnp.stack([layer["g1"], layer["b1"]]), 8)

        if li == 0:
            xn, stats2 = pl.pallas_call(
                _layer_first,
                grid=(2 * GSTEPS,),
                in_specs=[folded((BN, 8)), full2(emb_pad), folded((GB * K * NR, 8)),
                          full2(wcat_p), full2(wcat_q), full2(wrow),
                          full2(g1b1)],
                out_specs=layer_out_specs,
                out_shape=layer_out_shape,
                scratch_shapes=scratch,
            )(z3, emb_pad, sc, wcat_p, wcat_q, wrow, g1b1)
        else:
            xn, stats2 = pl.pallas_call(
                _layer_rest,
                grid=(2 * GSTEPS,),
                in_specs=[folded((BN, EMB)), full2(stats2), full2(g2b2),
                          folded((GB * K * NR, 8)), full2(wcat_p),
                          full2(wcat_q), full2(wrow), full2(g1b1)],
                out_specs=layer_out_specs,
                out_shape=layer_out_shape,
                scratch_shapes=scratch,
            )(xn, stats2, g2b2, sc, wcat_p, wcat_q, wrow, g1b1)
        g2b2 = _pad_rows(jnp.stack([layer["g2"], layer["b2"]]), 8)

    hw = _pad_rows(jnp.stack([
        params["W2"][:, 0],
        params["b1"],
        jnp.broadcast_to(params["b2"], (EMB,)),
    ]), 8)
    out = pl.pallas_call(
        _head_body,
        grid=(1,),
        in_specs=[pl.BlockSpec((G, NP, EMB), lambda g: (0, 0, 0)),
                  full2(stats2), full2(g2b2), full2(params["W1"]), full2(hw)],
        out_specs=pl.BlockSpec((G, 128), lambda g: (0, 0)),
        out_shape=jax.ShapeDtypeStruct((G, 128), f32),
    )(xn.reshape(G, NP, EMB), stats2, g2b2, params["W1"], hw)
    return out[:, 0]


# speedup vs baseline: 1.1340x; 1.1340x over previous
import functools

import jax
import jax.numpy as jnp
from jax.experimental import pallas as pl

G = 128
NPG = 78
N = G * NPG
EMB = 128
NUM_LAYERS = 4
K = 16
CUTOFF = 5.0
EPS = 1e-5
NP = 128
NR = 80
GB = 16
GSTEPS = G // GB
BN = GB * NP


def _build_body(posl_ref, poss_ref, sc_ref):
    i_row = jax.lax.broadcasted_iota(jnp.int32, (NP, 1), 0)
    j_col = jax.lax.broadcasted_iota(jnp.int32, (1, NP), 1)
    d2s = []
    for c in range(GB):
        px_r = posl_ref[0, 8 * c + 0, :].reshape(1, NP)
        py_r = posl_ref[0, 8 * c + 1, :].reshape(1, NP)
        pz_r = posl_ref[0, 8 * c + 2, :].reshape(1, NP)
        px_c = poss_ref[0, c * NP:(c + 1) * NP, 0:1]
        py_c = poss_ref[0, c * NP:(c + 1) * NP, 1:2]
        pz_c = poss_ref[0, c * NP:(c + 1) * NP, 2:3]
        dx = px_c - px_r
        dy = py_c - py_r
        dz = pz_c - pz_r
        d2c = (dx * dx + dy * dy) + dz * dz
        d2c = jnp.where(i_row == j_col, d2c + 1e9, d2c)
        d2s.append(d2c)
    d2 = jnp.concatenate(d2s, axis=0)
    jmat = jax.lax.broadcasted_iota(jnp.int32, (BN, NP), 1)
    jrow = jax.lax.broadcasted_iota(jnp.int32, (1, NP), 1)
    d2 = jnp.where(jrow >= NPG, 2e9, d2)
    ib = jax.lax.broadcasted_iota(jnp.int32, (BN, 1), 0)
    row_ok = (jnp.bitwise_and(ib, NP - 1) < NPG).astype(jnp.float32)
    cols, dists, invs = [], [], []
    for _ in range(K):
        mval = jnp.min(d2, axis=1, keepdims=True)
        idx = jnp.min(jnp.where(d2 == mval, jmat, jnp.int32(10**9)),
                      axis=1, keepdims=True)
        cols.append(idx.astype(jnp.float32))
        dists.append(jnp.sqrt(mval))
        invs.append(1.0 - jnp.where(mval <= CUTOFF * CUTOFF, row_ok, 0.0))
        d2 = jnp.where(jmat == idx, 2e9, d2)
    chunks = []
    for c in range(GB):
        r0 = c * NP
        for k in range(K):
            chunks.append(jnp.concatenate(
                [dists[k][r0:r0 + NR], invs[k][r0:r0 + NR],
                 cols[k][r0:r0 + NR], jnp.zeros((NR, 5), jnp.float32)],
                axis=1))
    sc_ref[0] = jnp.concatenate(chunks, axis=0)


def _bn_in(stats_ref, gb_ref):
    s = jnp.sum(stats_ref[...], axis=0)
    mu = s[0:1, :] / N
    var = s[1:2, :] / N - mu * mu
    rstd = jax.lax.rsqrt(var + EPS)
    return mu, rstd, gb_ref[0:1, :], gb_ref[1:2, :]


def _msg_core(xall, sc_ref, wp_ref, wq_ref, wrow_ref, agg_ref, stats_ref):
    pqp = jnp.dot(xall, wp_ref[...],
                  preferred_element_type=jnp.float32)
    pqq = jnp.dot(xall, wq_ref[...],
                  preferred_element_type=jnp.float32)
    jv = jax.lax.broadcasted_iota(jnp.int32, (1, NP), 1)
    wrow = wrow_ref[0:1, :]
    brow = wrow_ref[1:2, :]
    invrow = jnp.full((1, 256), -1e9, jnp.float32)
    s1 = jnp.zeros((1, EMB), jnp.float32)
    s2 = jnp.zeros((1, EMB), jnp.float32)
    EPG = K * NR
    for c in range(GB):
        r0 = c * NP
        pfs = pqp[r0:r0 + NR, :] + brow
        qfs = pqq[r0:r0 + NP, :]
        sc = sc_ref[0, c * EPG:(c + 1) * EPG, :]
        ohs = (sc[:, 2:3].astype(jnp.int32) == jv).astype(jnp.float32)
        comb = jnp.concatenate([ohs, sc[:, 0:2]], axis=1)
        table = jnp.concatenate([qfs, wrow, invrow], axis=0)
        zfs = jnp.dot(comb, table, preferred_element_type=jnp.float32)
        agg = jnp.zeros((NR, EMB), jnp.float32)
        for k in range(K):
            zk = zfs[k * NR:(k + 1) * NR, :] + pfs
            agg = agg + jax.nn.sigmoid(zk[:, 0:128]) \
                * jax.nn.softplus(zk[:, 128:256])
        agg_ref[0, r0:r0 + NR] = agg
        agg_ref[0, r0 + NR:r0 + NP] = jnp.zeros((NP - NR, EMB), jnp.float32)
        s1 = s1 + jnp.sum(agg, axis=0, keepdims=True)
        s2 = s2 + jnp.sum(agg * agg, axis=0, keepdims=True)
    stats_ref[0] = jnp.concatenate(
        [s1, s2, jnp.zeros((6, EMB), jnp.float32)], axis=0)


def _msg_first(z_ref, emb_ref, sc_ref, wp_ref, wq_ref, wrow_ref,
               agg_ref, stats_ref, x0_ref):
    zc = z_ref[0, :, 0:1]
    cls = jax.lax.broadcasted_iota(jnp.int32, (1, 128), 1)
    onehot = (zc == cls).astype(jnp.float32)
    xall = jnp.dot(onehot, emb_ref[...], preferred_element_type=jnp.float32)
    x0_ref[0] = xall
    _msg_core(xall, sc_ref, wp_ref, wq_ref, wrow_ref, agg_ref, stats_ref)


def _msg_rest(xn_ref, stats2_ref, g2b2_ref, sc_ref, wp_ref, wq_ref,
              wrow_ref, agg_ref, stats_ref):
    mu, rstd, g2, b2 = _bn_in(stats2_ref, g2b2_ref)
    ib = jax.lax.broadcasted_iota(jnp.int32, (BN, 1), 0)
    mask = (jnp.bitwise_and(ib, NP - 1) < NPG).astype(jnp.float32)
    xall = jax.nn.relu((xn_ref[0] - mu) * rstd * g2 + b2) * mask
    _msg_core(xall, sc_ref, wp_ref, wq_ref, wrow_ref, agg_ref, stats_ref)


def _bn1_body(first, xn_ref, stats2_ref, g2b2_ref, agg_ref, stats_ref,
              gb_ref, out_ref, ostats_ref):
    mu1, rstd1, g1, b1 = _bn_in(stats_ref, gb_ref)
    ib = jax.lax.broadcasted_iota(jnp.int32, (BN, 1), 0)
    mask = (jnp.bitwise_and(ib, NP - 1) < NPG).astype(jnp.float32)
    xn = xn_ref[0]
    if first:
        x = xn
    else:
        mu2, rstd2, g2, b2 = _bn_in(stats2_ref, g2b2_ref)
        x = jax.nn.relu((xn - mu2) * rstd2 * g2 + b2) * mask
    xo = (x + ((agg_ref[0] - mu1) * rstd1 * g1 + b1)) * mask
    out_ref[0] = xo
    s1 = jnp.sum(xo, axis=0, keepdims=True)
    s2 = jnp.sum(xo * xo, axis=0, keepdims=True)
    ostats_ref[0] = jnp.concatenate(
        [s1, s2, jnp.zeros((6, EMB), jnp.float32)], axis=0)


def _bn1_first(xn_ref, agg_ref, stats_ref, gb_ref, out_ref, ostats_ref):
    _bn1_body(True, xn_ref, None, None, agg_ref, stats_ref, gb_ref,
              out_ref, ostats_ref)


def _bn1_rest(xn_ref, stats2_ref, g2b2_ref, agg_ref, stats_ref, gb_ref,
              out_ref, ostats_ref):
    _bn1_body(False, xn_ref, stats2_ref, g2b2_ref, agg_ref, stats_ref,
              gb_ref, out_ref, ostats_ref)


def _head_body(xn_ref, stats2_ref, g2b2_ref, w1_ref, hw_ref, out_ref):
    mu, rstd, g2, b2 = _bn_in(stats2_ref, g2b2_ref)
    xn = xn_ref[...]
    ir = jax.lax.broadcasted_iota(jnp.int32, (1, NP, 1), 1)
    mask = (ir < NPG).astype(jnp.float32)
    x = jax.nn.relu((xn - mu[None]) * rstd[None] * g2[None] + b2[None]) * mask
    gmean = jnp.sum(x, axis=1) / NPG
    h = jax.nn.relu(
        jnp.dot(gmean, w1_ref[...], preferred_element_type=jnp.float32)
        + hw_ref[1:2, :])
    o = jnp.sum(h * hw_ref[0:1, :], axis=1, keepdims=True) + hw_ref[2:3, 0:1]
    out_ref[...] = jnp.broadcast_to(o, (G, 128))


def _pad_rows(a, rows):
    return jnp.pad(a, ((0, rows - a.shape[0]), (0, 0)))


@functools.partial(jax.jit, static_argnames=())
def kernel(z, pos, batch, params):
    f32 = jnp.float32
    posg = pos.reshape(G, NPG, 3)
    poss = jnp.pad(posg, ((0, 0), (0, NP - NPG), (0, 5))
                   ).reshape(GSTEPS, BN, 8)
    posl = jnp.pad(posg.transpose(0, 2, 1), ((0, 0), (0, 5), (0, NP - NPG))
                   ).reshape(GSTEPS, GB * 8, NP)
    z3 = jnp.pad(z.reshape(G, NPG, 1), ((0, 0), (0, NP - NPG), (0, 7))
                 ).reshape(GSTEPS, BN, 8)
    emb_pad = jnp.pad(params["emb"], ((0, 28), (0, 0)))

    gspec = lambda shape: pl.BlockSpec((1,) + shape, lambda g: (g, 0, 0))
    full2 = lambda a: pl.BlockSpec(a.shape, lambda g: (0,) * a.ndim)
    statspec = pl.BlockSpec((GSTEPS, 8, EMB), lambda g: (0, 0, 0))
    xspec = gspec((BN, EMB))
    scspec = gspec((GB * K * NR, 8))

    sc = pl.pallas_call(
        _build_body,
        grid=(GSTEPS,),
        in_specs=[gspec((GB * 8, NP)), gspec((BN, 8))],
        out_specs=scspec,
        out_shape=jax.ShapeDtypeStruct((GSTEPS, GB * K * NR, 8), f32),
    )(posl, poss)

    xn, stats2, g2b2 = None, None, None
    for li, layer in enumerate(params["layers"]):
        wf, ws = layer["Wf"], layer["Ws"]
        wcat_p = jnp.concatenate([wf[0:128], ws[0:128]], axis=1)
        wcat_q = jnp.concatenate([wf[128:256], ws[128:256]], axis=1)
        wrow = _pad_rows(jnp.stack([
            jnp.concatenate([wf[256], ws[256]]),
            jnp.concatenate([layer["bf"], layer["bs"]]),
        ]), 8)
        g1b1 = _pad_rows(jnp.stack([layer["g1"], layer["b1"]]), 8)

        if li == 0:
            out_specs = [xspec, gspec((8, EMB)), xspec]
            out_shape = [jax.ShapeDtypeStruct((GSTEPS, BN, EMB), f32),
                         jax.ShapeDtypeStruct((GSTEPS, 8, EMB), f32),
                         jax.ShapeDtypeStruct((GSTEPS, BN, EMB), f32)]
            agg, stats, xn = pl.pallas_call(
                _msg_first,
                grid=(GSTEPS,),
                in_specs=[gspec((BN, 8)), full2(emb_pad), scspec,
                          full2(wcat_p), full2(wcat_q), full2(wrow)],
                out_specs=out_specs,
                out_shape=out_shape,
            )(z3, emb_pad, sc, wcat_p, wcat_q, wrow)
        else:
            agg, stats = pl.pallas_call(
                _msg_rest,
                grid=(GSTEPS,),
                in_specs=[xspec, statspec, full2(g2b2), scspec,
                          full2(wcat_p), full2(wcat_q), full2(wrow)],
                out_specs=[xspec, gspec((8, EMB))],
                out_shape=[jax.ShapeDtypeStruct((GSTEPS, BN, EMB), f32),
                           jax.ShapeDtypeStruct((GSTEPS, 8, EMB), f32)],
            )(xn, stats2, g2b2, sc, wcat_p, wcat_q, wrow)

        bn_in = [xn] if li == 0 else [xn, stats2, g2b2]
        bn_specs = [xspec] if li == 0 else [xspec, statspec, full2(g2b2)]
        xn, stats2 = pl.pallas_call(
            _bn1_first if li == 0 else _bn1_rest,
            grid=(GSTEPS,),
            in_specs=bn_specs + [xspec, statspec, full2(g1b1)],
            out_specs=[xspec, gspec((8, EMB))],
            out_shape=[jax.ShapeDtypeStruct((GSTEPS, BN, EMB), f32),
                       jax.ShapeDtypeStruct((GSTEPS, 8, EMB), f32)],
        )(*bn_in, agg, stats, g1b1)
        g2b2 = _pad_rows(jnp.stack([layer["g2"], layer["b2"]]), 8)

    hw = _pad_rows(jnp.stack([
        params["W2"][:, 0],
        params["b1"],
        jnp.broadcast_to(params["b2"], (EMB,)),
    ]), 8)
    out = pl.pallas_call(
        _head_body,
        grid=(1,),
        in_specs=[pl.BlockSpec((G, NP, EMB), lambda g: (0, 0, 0)),
                  statspec, full2(g2b2), full2(params["W1"]), full2(hw)],
        out_specs=pl.BlockSpec((G, 128), lambda g: (0, 0)),
        out_shape=jax.ShapeDtypeStruct((G, 128), f32),
    )(xn.reshape(G, NP, EMB), stats2, g2b2, params["W1"], hw)
    return out[:, 0]
